# baseline (speedup 1.0000x reference)
import functools

import jax
import jax.numpy as jnp
from jax import lax
from jax.experimental import pallas as pl
from jax.experimental.pallas import tpu as pltpu
from jax.experimental.pallas import tpu_sc as plsc

NUM_EMB = 100000
DIM = 128
BATCH = 4096
HIST = 200
TOTAL = BATCH * HIST

NC, NS = 2, 16
NW = NC * NS


N_SEG = 4
SEG = TOTAL // N_SEG
CAST_BLK = 8192
SEG_BLKS = SEG // CAST_BLK


def _cast_seg_body(_, v_ref, o_ref):
    o_ref[...] = v_ref[...].astype(jnp.bfloat16)


def _cast_first_body(v_ref, o_ref):
    o_ref[...] = v_ref[...].astype(jnp.bfloat16)


def _cast_segment(seg, buf, rows_f32):
    out_spec = pl.BlockSpec(
        (CAST_BLK, DIM), lambda i, s=seg: (s * SEG_BLKS + i, 0)
    )
    rows_spec = pl.BlockSpec((CAST_BLK, DIM), lambda i: (i, 0))
    if buf is None:
        return pl.pallas_call(
            _cast_first_body,
            out_shape=jax.ShapeDtypeStruct((TOTAL, DIM), jnp.bfloat16),
            grid=(SEG_BLKS,),
            in_specs=[rows_spec],
            out_specs=out_spec,
        )(rows_f32)
    return pl.pallas_call(
        _cast_seg_body,
        out_shape=jax.ShapeDtypeStruct((TOTAL, DIM), jnp.bfloat16),
        grid=(SEG_BLKS,),
        in_specs=[
            pl.BlockSpec(memory_space=pl.ANY),
            rows_spec,
        ],
        out_specs=out_spec,
        input_output_aliases={0: 0},
    )(buf, rows_f32)


_mesh = plsc.VectorSubcoreMesh(core_axis_name="c", subcore_axis_name="s")

PER_W = SEG // NW
CHUNK = 128
NBUF = 5
N_STEPS = PER_W // (CHUNK * NBUF)


@functools.partial(
    pl.kernel,
    out_type=jax.ShapeDtypeStruct((SEG, DIM), jnp.float32),
    mesh=_mesh,
    scratch_types=[
        pltpu.VMEM((PER_W,), jnp.int32),
        pltpu.VMEM((NBUF, CHUNK, DIM), jnp.float32),
        pltpu.SemaphoreType.DMA,
        pltpu.SemaphoreType.DMA,
    ],
)
def _sc_gather(table_hbm, idx_hbm, out_hbm, idx_v, rows_v, gsem, ssem):
    wid = lax.axis_index("s") * NC + lax.axis_index("c")
    base = wid * PER_W
    pltpu.sync_copy(idx_hbm.at[pl.ds(base, PER_W)], idx_v)

    def body(i, carry):
        step = i * (CHUNK * NBUF)
        gathers = []
        for b in range(NBUF):
            off = step + b * CHUNK
            gathers.append(
                pltpu.async_copy(
                    table_hbm.at[idx_v.at[pl.ds(off, CHUNK)]],
                    rows_v.at[b],
                    gsem,
                )
            )
        stores = []
        for b in range(NBUF):
            off = step + b * CHUNK
            gathers[b].wait()
            stores.append(
                pltpu.async_copy(
                    rows_v.at[b],
                    out_hbm.at[pl.ds(base + off, CHUNK)],
                    ssem,
                )
            )
        for st in stores:
            st.wait()
        return carry

    lax.fori_loop(0, N_STEPS, body, 0)


def kernel(x, weight):
    idx_flat = x.reshape(TOTAL)
    buf = None
    for s in range(N_SEG):
        rows_f32 = _sc_gather(weight, idx_flat[s * SEG:(s + 1) * SEG])
        buf = _cast_segment(s, buf, rows_f32)
    return buf.reshape(BATCH, HIST, DIM)
